# baseline (speedup 1.0000x reference)
import jax
import jax.numpy as jnp
from jax.experimental import pallas as pl
from jax.experimental.pallas import tpu as pltpu

S, B, E, H1, H2 = 512, 256, 300, 16, 64
TC = 32
TD = 32
NBC = S // TC + 1
NBD = 17
NI = NBC * TC
BF = jnp.bfloat16


def _gru(xd, hd, bias, h, hdim):
    r = jax.nn.sigmoid(xd[:hdim] + (hd[:hdim] + bias[:hdim]))
    z = jax.nn.sigmoid(xd[hdim:2 * hdim]
                       + (hd[hdim:2 * hdim] + bias[hdim:2 * hdim]))
    n = jnp.tanh((xd[2 * hdim:] + bias[2 * hdim:3 * hdim])
                 + r * (hd[2 * hdim:] + bias[3 * hdim:]))
    return n + z * (h - n)


def _dot(a, b):
    return jnp.dot(a, b, preferred_element_type=jnp.float32)


def _c1_kernel(x_ref, w1_ref, b1_ref, u1_ref,
               w2_ref, b2_ref, u2_ref,
               y_ref, h1_ref, h2_ref):
    blk = pl.program_id(0)

    def block(mode):
        if mode == "lower":
            h1 = jnp.zeros_like(h1_ref)
            h2 = jnp.zeros_like(h2_ref)
        else:
            h1 = h1_ref[...]
            h2 = h2_ref[...]
        for dt in range(TC):
            h1b = h1.astype(BF)
            h2b = h2.astype(BF)
            xd1 = jax.lax.dot_general(
                w1_ref[...], x_ref[dt].astype(BF), (((1,), (1,)), ((), ())),
                preferred_element_type=jnp.float32)
            hd1 = _dot(u1_ref[...], h1b)
            h1n = _gru(xd1, hd1, b1_ref[...], h1, H1)
            xd2 = _dot(w2_ref[...], h1b)
            hd2 = _dot(u2_ref[...], h2b)
            h2n = _gru(xd2, hd2, b2_ref[...], h2, H1)
            if mode == "lower":
                i = blk * TC + dt
                h1 = h1n
                h2 = jnp.where(i >= 1, h2n, h2)
            elif mode == "upper":
                i = blk * TC + dt
                h1 = jnp.where(i < S, h1n, h1)
                h2 = jnp.where(i < S + 1, h2n, h2)
            else:
                h1 = h1n
                h2 = h2n
            y_ref[dt] = h2n.astype(BF)
        h1_ref[...] = h1
        h2_ref[...] = h2

    @pl.when(blk == 0)
    def _():
        block("lower")

    @pl.when(jnp.logical_and(blk > 0, blk < NBC - 1))
    def _():
        block("mid")

    @pl.when(blk == NBC - 1)
    def _():
        block("upper")


def _dec_kernel(y_ref, h10_ref, h11_ref,
                w3_ref, b3_ref, u3_ref,
                w4_ref, b4_ref, u4_ref,
                w5_ref, b5_ref, u5_ref,
                w6_ref, b6_ref, u6_ref,
                w7_ref, b7_ref, u7_ref,
                out_ref,
                h3_ref, h4_ref, h5_ref, h6_ref, h7_ref):
    blk = pl.program_id(0)

    def block(mode):
        if mode == "lower":
            h3 = h10_ref[...]
            h4 = h11_ref[...]
            h5 = jnp.zeros_like(h5_ref)
            h6 = jnp.zeros_like(h6_ref)
            h7 = jnp.zeros_like(h7_ref)
        else:
            h3 = h3_ref[...]
            h4 = h4_ref[...]
            h5 = h5_ref[...]
            h6 = h6_ref[...]
            h7 = h7_ref[...]
        for dt in range(TD):
            h3b = h3.astype(BF)
            h4b = h4.astype(BF)
            h5b = h5.astype(BF)
            h6b = h6.astype(BF)
            h7b = h7.astype(BF)
            xd = _dot(w3_ref[...], y_ref[dt])
            hd = _dot(u3_ref[...], h3b)
            h3n = _gru(xd, hd, b3_ref[...], h3, H1)

            xd = _dot(w4_ref[...], h3b)
            hd = _dot(u4_ref[...], h4b)
            h4n = _gru(xd, hd, b4_ref[...], h4, H1)

            xd = _dot(w5_ref[...], h4b)
            hd = _dot(u5_ref[...], h5b)
            h5n = _gru(xd, hd, b5_ref[...], h5, H2)

            xd = _dot(w6_ref[...], h5b)
            hd = _dot(u6_ref[...], h6b)
            h6n = _gru(xd, hd, b6_ref[...], h6, H2)

            xd = _dot(w7_ref[...], h6b)
            hd = _dot(u7_ref[...], h7b)
            h7n = _gru(xd, hd, b7_ref[...], h7, H2)

            if mode == "lower":
                j = blk * TD + dt
                h3 = jnp.where(j >= 1, h3n, h3)
                h4 = jnp.where(j >= 2, h4n, h4)
                h5 = jnp.where(j >= 3, h5n, h5)
                h6 = jnp.where(j >= 4, h6n, h6)
                h7 = jnp.where(j >= 5, h7n, h7)
            elif mode == "upper":
                j = blk * TD + dt
                h3 = jnp.where(j < S + 1, h3n, h3)
                h4 = jnp.where(j < S + 2, h4n, h4)
                h5 = jnp.where(j < S + 3, h5n, h5)
                h6 = jnp.where(j < S + 4, h6n, h6)
                h7 = jnp.where(j < S + 5, h7n, h7)
            else:
                h3, h4, h5, h6, h7 = h3n, h4n, h5n, h6n, h7n
        h3_ref[...] = h3
        h4_ref[...] = h4
        h5_ref[...] = h5
        h6_ref[...] = h6
        h7_ref[...] = h7
        if mode == "upper":
            out_ref[:H2, :] = h7
            out_ref[H2:, :] = h6

    @pl.when(blk == 0)
    def _():
        block("lower")

    @pl.when(jnp.logical_and(blk > 0, blk < NBD - 1))
    def _():
        block("mid")

    @pl.when(blk == NBD - 1)
    def _():
        block("upper")


def _full2(shape):
    return pl.BlockSpec(shape, lambda i: (0, 0))


def kernel(x,
           c1_wih0, c1_whh0, c1_bih0, c1_bhh0,
           c1_wih1, c1_whh1, c1_bih1, c1_bhh1,
           d1_wih0, d1_whh0, d1_bih0, d1_bhh0,
           d1_wih1, d1_whh1, d1_bih1, d1_bhh1,
           d2_wih0, d2_whh0, d2_bih0, d2_bhh0,
           d2_wih1, d2_whh1, d2_bih1, d2_bhh1,
           d2_wih2, d2_whh2, d2_bih2, d2_bhh2):
    f32 = jnp.float32

    def pb(bih, bhh):
        hdim = bih.shape[0] // 3
        packed = jnp.concatenate([
            bih[:2 * hdim] + bhh[:2 * hdim],
            bih[2 * hdim:],
            bhh[2 * hdim:],
        ])
        return jnp.broadcast_to(packed.reshape(-1, 1), (4 * hdim, B))

    def wb(w):
        return w.astype(BF)

    y_pad, h10, h11 = pl.pallas_call(
        _c1_kernel,
        grid=(NBC,),
        in_specs=[
            pl.BlockSpec((TC, B, E), lambda i: (jnp.minimum(i, S // TC - 1), 0, 0)),
            _full2((3 * H1, E)), _full2((4 * H1, B)), _full2((3 * H1, H1)),
            _full2((3 * H1, H1)), _full2((4 * H1, B)), _full2((3 * H1, H1)),
        ],
        out_specs=[
            pl.BlockSpec((TC, H1, B), lambda i: (i, 0, 0)),
            pl.BlockSpec((H1, B), lambda i: (0, 0)),
            pl.BlockSpec((H1, B), lambda i: (0, 0)),
        ],
        out_shape=[
            jax.ShapeDtypeStruct((NI, H1, B), BF),
            jax.ShapeDtypeStruct((H1, B), f32),
            jax.ShapeDtypeStruct((H1, B), f32),
        ],
        compiler_params=pltpu.CompilerParams(
            dimension_semantics=("arbitrary",),
        ),
        name="sentemb_compress1",
    )(x, wb(c1_wih0), pb(c1_bih0, c1_bhh0), wb(c1_whh0),
      wb(c1_wih1), pb(c1_bih1, c1_bhh1), wb(c1_whh1))

    outT = pl.pallas_call(
        _dec_kernel,
        grid=(NBD,),
        in_specs=[
            pl.BlockSpec((TD, H1, B), lambda i: (i, 0, 0)),
            pl.BlockSpec((H1, B), lambda i: (0, 0)),
            pl.BlockSpec((H1, B), lambda i: (0, 0)),
            _full2((3 * H1, H1)), _full2((4 * H1, B)), _full2((3 * H1, H1)),
            _full2((3 * H1, H1)), _full2((4 * H1, B)), _full2((3 * H1, H1)),
            _full2((3 * H2, H1)), _full2((4 * H2, B)), _full2((3 * H2, H2)),
            _full2((3 * H2, H2)), _full2((4 * H2, B)), _full2((3 * H2, H2)),
            _full2((3 * H2, H2)), _full2((4 * H2, B)), _full2((3 * H2, H2)),
        ],
        out_specs=pl.BlockSpec((2 * H2, B), lambda i: (0, 0)),
        out_shape=jax.ShapeDtypeStruct((2 * H2, B), f32),
        scratch_shapes=[
            pltpu.VMEM((H1, B), f32),
            pltpu.VMEM((H1, B), f32),
            pltpu.VMEM((H2, B), f32),
            pltpu.VMEM((H2, B), f32),
            pltpu.VMEM((H2, B), f32),
        ],
        compiler_params=pltpu.CompilerParams(
            dimension_semantics=("arbitrary",),
        ),
        name="sentemb_decode",
    )(y_pad, h10, h11,
      wb(d1_wih0), pb(d1_bih0, d1_bhh0), wb(d1_whh0),
      wb(d1_wih1), pb(d1_bih1, d1_bhh1), wb(d1_whh1),
      wb(d2_wih0), pb(d2_bih0, d2_bhh0), wb(d2_whh0),
      wb(d2_wih1), pb(d2_bih1, d2_bhh1), wb(d2_whh1),
      wb(d2_wih2), pb(d2_bih2, d2_bhh2), wb(d2_whh2))

    return outT.T


# speedup vs baseline: 1.0043x; 1.0043x over previous
import jax
import jax.numpy as jnp
from jax.experimental import pallas as pl
from jax.experimental.pallas import tpu as pltpu

S, B, E, H1, H2 = 512, 256, 300, 16, 64
TC = 32
TD = 16
NBC = S // TC + 1
NBD = 33
NI = NBC * TC
BF = jnp.bfloat16


def _gru(xd, hd, bias, h, hdim):
    r = jax.nn.sigmoid(xd[:hdim] + (hd[:hdim] + bias[:hdim]))
    z = jax.nn.sigmoid(xd[hdim:2 * hdim]
                       + (hd[hdim:2 * hdim] + bias[hdim:2 * hdim]))
    n = jnp.tanh((xd[2 * hdim:] + bias[2 * hdim:3 * hdim])
                 + r * (hd[2 * hdim:] + bias[3 * hdim:]))
    return n + z * (h - n)


def _dot(a, b):
    return jnp.dot(a, b, preferred_element_type=jnp.float32)


def _c1_kernel(x_ref, w1_ref, b1_ref, u1_ref,
               w2_ref, b2_ref, u2_ref,
               y_ref, h1_ref, h2_ref):
    blk = pl.program_id(0)

    def block(mode):
        if mode == "lower":
            h1 = jnp.zeros_like(h1_ref)
            h2 = jnp.zeros_like(h2_ref)
        else:
            h1 = h1_ref[...]
            h2 = h2_ref[...]
        for dt in range(TC):
            h1b = h1.astype(BF)
            h2b = h2.astype(BF)
            xd1 = jax.lax.dot_general(
                w1_ref[...], x_ref[dt].astype(BF), (((1,), (1,)), ((), ())),
                preferred_element_type=jnp.float32)
            hd1 = _dot(u1_ref[...], h1b)
            h1n = _gru(xd1, hd1, b1_ref[...], h1, H1)
            xd2 = _dot(w2_ref[...], h1b)
            hd2 = _dot(u2_ref[...], h2b)
            h2n = _gru(xd2, hd2, b2_ref[...], h2, H1)
            if mode == "lower":
                i = blk * TC + dt
                h1 = h1n
                h2 = jnp.where(i >= 1, h2n, h2)
            elif mode == "upper":
                i = blk * TC + dt
                h1 = jnp.where(i < S, h1n, h1)
                h2 = jnp.where(i < S + 1, h2n, h2)
            else:
                h1 = h1n
                h2 = h2n
            y_ref[dt] = h2n.astype(BF)
        h1_ref[...] = h1
        h2_ref[...] = h2

    @pl.when(blk == 0)
    def _():
        block("lower")

    @pl.when(jnp.logical_and(blk > 0, blk < NBC - 1))
    def _():
        block("mid")

    @pl.when(blk == NBC - 1)
    def _():
        block("upper")


def _dec_kernel(y_ref, h10_ref, h11_ref,
                w3_ref, b3_ref, u3_ref,
                w4_ref, b4_ref, u4_ref,
                w5_ref, b5_ref, u5_ref,
                w6_ref, b6_ref, u6_ref,
                w7_ref, b7_ref, u7_ref,
                out_ref,
                h3_ref, h4_ref, h5_ref, h6_ref, h7_ref):
    blk = pl.program_id(0)

    def block(mode):
        if mode == "lower":
            h3 = h10_ref[...]
            h4 = h11_ref[...]
            h5 = jnp.zeros_like(h5_ref)
            h6 = jnp.zeros_like(h6_ref)
            h7 = jnp.zeros_like(h7_ref)
        else:
            h3 = h3_ref[...]
            h4 = h4_ref[...]
            h5 = h5_ref[...]
            h6 = h6_ref[...]
            h7 = h7_ref[...]
        for dt in range(TD):
            h3b = h3.astype(BF)
            h4b = h4.astype(BF)
            h5b = h5.astype(BF)
            h6b = h6.astype(BF)
            h7b = h7.astype(BF)
            xd = _dot(w3_ref[...], y_ref[dt])
            hd = _dot(u3_ref[...], h3b)
            h3n = _gru(xd, hd, b3_ref[...], h3, H1)

            xd = _dot(w4_ref[...], h3b)
            hd = _dot(u4_ref[...], h4b)
            h4n = _gru(xd, hd, b4_ref[...], h4, H1)

            xd = _dot(w5_ref[...], h4b)
            hd = _dot(u5_ref[...], h5b)
            h5n = _gru(xd, hd, b5_ref[...], h5, H2)

            xd = _dot(w6_ref[...], h5b)
            hd = _dot(u6_ref[...], h6b)
            h6n = _gru(xd, hd, b6_ref[...], h6, H2)

            xd = _dot(w7_ref[...], h6b)
            hd = _dot(u7_ref[...], h7b)
            h7n = _gru(xd, hd, b7_ref[...], h7, H2)

            if mode == "lower":
                j = blk * TD + dt
                h3 = jnp.where(j >= 1, h3n, h3)
                h4 = jnp.where(j >= 2, h4n, h4)
                h5 = jnp.where(j >= 3, h5n, h5)
                h6 = jnp.where(j >= 4, h6n, h6)
                h7 = jnp.where(j >= 5, h7n, h7)
            elif mode == "upper":
                j = blk * TD + dt
                h3 = jnp.where(j < S + 1, h3n, h3)
                h4 = jnp.where(j < S + 2, h4n, h4)
                h5 = jnp.where(j < S + 3, h5n, h5)
                h6 = jnp.where(j < S + 4, h6n, h6)
                h7 = jnp.where(j < S + 5, h7n, h7)
            else:
                h3, h4, h5, h6, h7 = h3n, h4n, h5n, h6n, h7n
        h3_ref[...] = h3
        h4_ref[...] = h4
        h5_ref[...] = h5
        h6_ref[...] = h6
        h7_ref[...] = h7
        if mode == "upper":
            out_ref[:H2, :] = h7
            out_ref[H2:, :] = h6

    @pl.when(blk == 0)
    def _():
        block("lower")

    @pl.when(jnp.logical_and(blk > 0, blk < NBD - 1))
    def _():
        block("mid")

    @pl.when(blk == NBD - 1)
    def _():
        block("upper")


def _full2(shape):
    return pl.BlockSpec(shape, lambda i: (0, 0))


def kernel(x,
           c1_wih0, c1_whh0, c1_bih0, c1_bhh0,
           c1_wih1, c1_whh1, c1_bih1, c1_bhh1,
           d1_wih0, d1_whh0, d1_bih0, d1_bhh0,
           d1_wih1, d1_whh1, d1_bih1, d1_bhh1,
           d2_wih0, d2_whh0, d2_bih0, d2_bhh0,
           d2_wih1, d2_whh1, d2_bih1, d2_bhh1,
           d2_wih2, d2_whh2, d2_bih2, d2_bhh2):
    f32 = jnp.float32

    def pb(bih, bhh):
        hdim = bih.shape[0] // 3
        packed = jnp.concatenate([
            bih[:2 * hdim] + bhh[:2 * hdim],
            bih[2 * hdim:],
            bhh[2 * hdim:],
        ])
        return jnp.broadcast_to(packed.reshape(-1, 1), (4 * hdim, B))

    def wb(w):
        return w.astype(BF)

    y_pad, h10, h11 = pl.pallas_call(
        _c1_kernel,
        grid=(NBC,),
        in_specs=[
            pl.BlockSpec((TC, B, E), lambda i: (jnp.minimum(i, S // TC - 1), 0, 0)),
            _full2((3 * H1, E)), _full2((4 * H1, B)), _full2((3 * H1, H1)),
            _full2((3 * H1, H1)), _full2((4 * H1, B)), _full2((3 * H1, H1)),
        ],
        out_specs=[
            pl.BlockSpec((TC, H1, B), lambda i: (i, 0, 0)),
            pl.BlockSpec((H1, B), lambda i: (0, 0)),
            pl.BlockSpec((H1, B), lambda i: (0, 0)),
        ],
        out_shape=[
            jax.ShapeDtypeStruct((NI, H1, B), BF),
            jax.ShapeDtypeStruct((H1, B), f32),
            jax.ShapeDtypeStruct((H1, B), f32),
        ],
        compiler_params=pltpu.CompilerParams(
            dimension_semantics=("arbitrary",),
        ),
        name="sentemb_compress1",
    )(x, wb(c1_wih0), pb(c1_bih0, c1_bhh0), wb(c1_whh0),
      wb(c1_wih1), pb(c1_bih1, c1_bhh1), wb(c1_whh1))

    outT = pl.pallas_call(
        _dec_kernel,
        grid=(NBD,),
        in_specs=[
            pl.BlockSpec((TD, H1, B), lambda i: (i, 0, 0)),
            pl.BlockSpec((H1, B), lambda i: (0, 0)),
            pl.BlockSpec((H1, B), lambda i: (0, 0)),
            _full2((3 * H1, H1)), _full2((4 * H1, B)), _full2((3 * H1, H1)),
            _full2((3 * H1, H1)), _full2((4 * H1, B)), _full2((3 * H1, H1)),
            _full2((3 * H2, H1)), _full2((4 * H2, B)), _full2((3 * H2, H2)),
            _full2((3 * H2, H2)), _full2((4 * H2, B)), _full2((3 * H2, H2)),
            _full2((3 * H2, H2)), _full2((4 * H2, B)), _full2((3 * H2, H2)),
        ],
        out_specs=pl.BlockSpec((2 * H2, B), lambda i: (0, 0)),
        out_shape=jax.ShapeDtypeStruct((2 * H2, B), f32),
        scratch_shapes=[
            pltpu.VMEM((H1, B), f32),
            pltpu.VMEM((H1, B), f32),
            pltpu.VMEM((H2, B), f32),
            pltpu.VMEM((H2, B), f32),
            pltpu.VMEM((H2, B), f32),
        ],
        compiler_params=pltpu.CompilerParams(
            dimension_semantics=("arbitrary",),
        ),
        name="sentemb_decode",
    )(y_pad, h10, h11,
      wb(d1_wih0), pb(d1_bih0, d1_bhh0), wb(d1_whh0),
      wb(d1_wih1), pb(d1_bih1, d1_bhh1), wb(d1_whh1),
      wb(d2_wih0), pb(d2_bih0, d2_bhh0), wb(d2_whh0),
      wb(d2_wih1), pb(d2_bih1, d2_bhh1), wb(d2_whh1),
      wb(d2_wih2), pb(d2_bih2, d2_bhh2), wb(d2_whh2))

    return outT.T
